# Initial kernel scaffold; baseline (speedup 1.0000x reference)
import jax
import jax.numpy as jnp
from jax.experimental import pallas as pl


def kernel(x, importance, W_proj, b_proj, neuron_emb):
    raise NotImplementedError("write your pallas kernel here")


# speedup vs baseline: 1.2538x; 1.2538x over previous
import functools

import jax
import jax.numpy as jnp
from jax.experimental import pallas as pl

D_MODEL = 1024
D_SPACE = 64
N_FQK = 2048
N_FV = 1024
N_REL = 512
N_VAL = 256
N_USED = N_FQK + N_FV + N_REL + N_VAL
TOPK_QK = 64
TOPK_V = 32
TILE_S = 512

_SLICES = (
    (0, N_FQK),
    (N_FQK, N_FV),
    (N_FQK + N_FV, N_REL),
    (N_FQK + N_FV + N_REL, N_VAL),
)


def _topk_sorted_idx_col(w, k):
    n = w.shape[1]
    iota = jax.lax.broadcasted_iota(jnp.int32, (1, n), 1)

    def body(_, wcur):
        m = jnp.max(wcur)
        sel = jnp.min(jnp.where(wcur == m, iota, n))
        return jnp.where(iota == sel, jnp.float32(-1e30), wcur)

    wmask = jax.lax.fori_loop(0, k, body, w)
    mask = wmask < -1e29

    p = mask.astype(jnp.int32)
    shift = 1
    while shift < n:
        shifted = jnp.concatenate(
            [jnp.zeros((1, shift), jnp.int32), p[:, : n - shift]], axis=1
        )
        p = p + shifted
        shift *= 2
    pos = p - 1

    posb = jnp.broadcast_to(pos, (k, n))
    maskb = jnp.broadcast_to(mask, (k, n))
    prow = jax.lax.broadcasted_iota(jnp.int32, (k, n), 0)
    lane = jax.lax.broadcasted_iota(jnp.int32, (k, n), 1)
    contrib = jnp.where(maskb & (posb == prow), lane, 0)
    return jnp.sum(contrib, axis=1, keepdims=True)


def _router_body(x_ref, imp_ref, w_ref, b_ref, emb_ref,
                 out_w_ref, out_qk_ref, out_v_ref):
    s = pl.program_id(1)
    ns = pl.num_programs(1)

    xt = x_ref[0]
    h = jax.lax.dot_general(
        xt, w_ref[...], (((1,), (1,)), ((), ())),
        preferred_element_type=jnp.float32,
    ) + b_ref[...]

    emb = emb_ref[...]
    nrm = jnp.sqrt(jnp.sum(emb * emb, axis=1, keepdims=True))
    emb_n = emb / jnp.maximum(nrm, 1e-12)
    logits = jax.lax.dot_general(
        h, emb_n, (((1,), (1,)), ((), ())),
        preferred_element_type=jnp.float32,
    )

    imp_col = imp_ref[0]

    parts = []
    for start, width in _SLICES:
        sl = logits[:, start:start + width]
        m = jnp.max(sl, axis=1, keepdims=True)
        e = jnp.exp(sl - m)
        denom = jnp.sum(e, axis=1, keepdims=True)
        parts.append(jnp.sum(e * (imp_col / denom), axis=0, keepdims=True))
    partial = jnp.concatenate(parts, axis=1)

    @pl.when(s == 0)
    def _():
        out_w_ref[0] = partial

    @pl.when(s != 0)
    def _():
        out_w_ref[0] += partial

    @pl.when(s == ns - 1)
    def _():
        wfull = out_w_ref[0]
        out_qk_ref[0] = _topk_sorted_idx_col(wfull[:, :N_FQK], TOPK_QK)
        out_v_ref[0] = _topk_sorted_idx_col(
            wfull[:, N_FQK:N_FQK + N_FV], TOPK_V)


@functools.partial(jax.jit, static_argnames=())
def kernel(x, importance, W_proj, b_proj, neuron_emb):
    B, S, _ = x.shape
    imp3 = importance.reshape(B, S, 1)
    b2 = b_proj.reshape(1, D_SPACE)
    emb_used = neuron_emb[:N_USED]

    grid = (B, S // TILE_S)
    w3, qk3, v3 = pl.pallas_call(
        _router_body,
        grid=grid,
        in_specs=[
            pl.BlockSpec((1, TILE_S, D_MODEL), lambda b, s: (b, s, 0)),
            pl.BlockSpec((1, TILE_S, 1), lambda b, s: (b, s, 0)),
            pl.BlockSpec((D_SPACE, D_MODEL), lambda b, s: (0, 0)),
            pl.BlockSpec((1, D_SPACE), lambda b, s: (0, 0)),
            pl.BlockSpec((N_USED, D_SPACE), lambda b, s: (0, 0)),
        ],
        out_specs=[
            pl.BlockSpec((1, 1, N_USED), lambda b, s: (b, 0, 0)),
            pl.BlockSpec((1, TOPK_QK, 1), lambda b, s: (b, 0, 0)),
            pl.BlockSpec((1, TOPK_V, 1), lambda b, s: (b, 0, 0)),
        ],
        out_shape=[
            jax.ShapeDtypeStruct((B, 1, N_USED), jnp.float32),
            jax.ShapeDtypeStruct((B, TOPK_QK, 1), jnp.int32),
            jax.ShapeDtypeStruct((B, TOPK_V, 1), jnp.int32),
        ],
    )(x, imp3, W_proj, b2, emb_used)

    weights = w3.reshape(B, N_USED)
    idx_qk = qk3.reshape(B, TOPK_QK)
    idx_v = v3.reshape(B, TOPK_V)
    rel = weights[:, N_FQK + N_FV:N_FQK + N_FV + N_REL]
    val = weights[:, N_FQK + N_FV + N_REL:]
    return (idx_qk, idx_v, rel, rel, val)


# speedup vs baseline: 1.3881x; 1.1071x over previous
import functools

import jax
import jax.numpy as jnp
from jax.experimental import pallas as pl
from jax.experimental.pallas import tpu as pltpu

D_MODEL = 1024
D_SPACE = 64
N_FQK = 2048
N_FV = 1024
N_REL = 512
N_VAL = 256
N_USED = N_FQK + N_FV + N_REL + N_VAL
TOPK_QK = 64
TOPK_V = 32
TILE_S = 512

_SLICES = (
    (0, N_FQK),
    (N_FQK, N_FV),
    (N_FQK + N_FV, N_REL),
    (N_FQK + N_FV + N_REL, N_VAL),
)


def _topk_sorted_idx_col(w, k):
    n = w.shape[1]
    iota = jax.lax.broadcasted_iota(jnp.int32, (1, n), 1)

    def body(_, wcur):
        m = jnp.max(wcur)
        sel = jnp.min(jnp.where(wcur == m, iota, n))
        return jnp.where(iota == sel, jnp.float32(-1e30), wcur)

    wmask = jax.lax.fori_loop(0, k, body, w)
    mask = wmask < -1e29

    p = mask.astype(jnp.int32)
    shift = 1
    while shift < n:
        shifted = jnp.concatenate(
            [jnp.zeros((1, shift), jnp.int32), p[:, : n - shift]], axis=1
        )
        p = p + shifted
        shift *= 2
    pos = p - 1

    posb = jnp.broadcast_to(pos, (k, n))
    maskb = jnp.broadcast_to(mask, (k, n))
    prow = jax.lax.broadcasted_iota(jnp.int32, (k, n), 0)
    lane = jax.lax.broadcasted_iota(jnp.int32, (k, n), 1)
    contrib = jnp.where(maskb & (posb == prow), lane, 0)
    return jnp.sum(contrib, axis=1, keepdims=True)


def _router_body(x_ref, imp_ref, w_ref, b_ref, emb_ref,
                 out_w_ref, out_qk_ref, out_v_ref, embn_ref):
    b = pl.program_id(0)
    s = pl.program_id(1)
    ns = pl.num_programs(1)

    @pl.when((b == 0) & (s == 0))
    def _():
        emb = emb_ref[...]
        nrm = jnp.sqrt(jnp.sum(emb * emb, axis=1, keepdims=True))
        embn_ref[...] = emb * (1.0 / jnp.maximum(nrm, 1e-12))

    xt = x_ref[0]
    h = jax.lax.dot_general(
        xt, w_ref[...], (((1,), (1,)), ((), ())),
        preferred_element_type=jnp.float32,
    ) + b_ref[...]

    logits = jax.lax.dot_general(
        h, embn_ref[...], (((1,), (1,)), ((), ())),
        preferred_element_type=jnp.float32,
    )

    imp_col = imp_ref[0]

    parts = []
    for start, width in _SLICES:
        sl = logits[:, start:start + width]
        m = jnp.max(sl, axis=1, keepdims=True)
        e = jnp.exp(sl - m)
        denom = jnp.sum(e, axis=1, keepdims=True)
        c = imp_col / denom
        parts.append(jax.lax.dot_general(
            c, e, (((0,), (0,)), ((), ())),
            preferred_element_type=jnp.float32))
    partial = jnp.concatenate(parts, axis=1)

    @pl.when(s == 0)
    def _():
        out_w_ref[0] = partial

    @pl.when(s != 0)
    def _():
        out_w_ref[0] += partial

    @pl.when(s == ns - 1)
    def _():
        wfull = out_w_ref[0]
        out_qk_ref[0] = _topk_sorted_idx_col(wfull[:, :N_FQK], TOPK_QK)
        out_v_ref[0] = _topk_sorted_idx_col(
            wfull[:, N_FQK:N_FQK + N_FV], TOPK_V)


@functools.partial(jax.jit, static_argnames=())
def kernel(x, importance, W_proj, b_proj, neuron_emb):
    B, S, _ = x.shape
    imp3 = importance.reshape(B, S, 1)
    b2 = b_proj.reshape(1, D_SPACE)
    emb_used = neuron_emb[:N_USED]

    grid = (B, S // TILE_S)
    w3, qk3, v3 = pl.pallas_call(
        _router_body,
        grid=grid,
        in_specs=[
            pl.BlockSpec((1, TILE_S, D_MODEL), lambda b, s: (b, s, 0)),
            pl.BlockSpec((1, TILE_S, 1), lambda b, s: (b, s, 0)),
            pl.BlockSpec((D_SPACE, D_MODEL), lambda b, s: (0, 0)),
            pl.BlockSpec((1, D_SPACE), lambda b, s: (0, 0)),
            pl.BlockSpec((N_USED, D_SPACE), lambda b, s: (0, 0)),
        ],
        out_specs=[
            pl.BlockSpec((1, 1, N_USED), lambda b, s: (b, 0, 0)),
            pl.BlockSpec((1, TOPK_QK, 1), lambda b, s: (b, 0, 0)),
            pl.BlockSpec((1, TOPK_V, 1), lambda b, s: (b, 0, 0)),
        ],
        out_shape=[
            jax.ShapeDtypeStruct((B, 1, N_USED), jnp.float32),
            jax.ShapeDtypeStruct((B, TOPK_QK, 1), jnp.int32),
            jax.ShapeDtypeStruct((B, TOPK_V, 1), jnp.int32),
        ],
        scratch_shapes=[pltpu.VMEM((N_USED, D_SPACE), jnp.float32)],
    )(x, imp3, W_proj, b2, emb_used)

    weights = w3.reshape(B, N_USED)
    idx_qk = qk3.reshape(B, TOPK_QK)
    idx_v = v3.reshape(B, TOPK_V)
    rel = weights[:, N_FQK + N_FV:N_FQK + N_FV + N_REL]
    val = weights[:, N_FQK + N_FV + N_REL:]
    return (idx_qk, idx_v, rel, rel, val)
